# baseline (speedup 1.0000x reference)
import functools

import jax
import jax.numpy as jnp
from jax import lax
from jax.experimental import pallas as pl
from jax.experimental.pallas import tpu as pltpu
from jax.experimental.pallas import tpu_sc as plsc

VOCAB = 1000000
HIDDEN = 64
B = 4096
L = 200

NC = 2
NS = 16
NW = NC * NS

N = B * L
CH = 128
N_BLK = N // CH
BLK_W = N_BLK // NW

_mesh = plsc.VectorSubcoreMesh(
    core_axis_name="c", subcore_axis_name="s", num_cores=NC, num_subcores=NS
)


@functools.partial(
    pl.kernel,
    out_type=jax.ShapeDtypeStruct((N, HIDDEN), jnp.float32),
    mesh=_mesh,
    scratch_types=[
        pltpu.VMEM((BLK_W, CH), jnp.int32),
        pltpu.VMEM((CH, HIDDEN), jnp.float32),
        pltpu.SemaphoreType.DMA,
    ],
    compiler_params=pltpu.CompilerParams(use_tc_tiling_on_sc=False),
)
def _embed_lookup(ids_hbm, table_hbm, out_hbm, idx_v, row_v, sem):
    wid = lax.axis_index("s") * NC + lax.axis_index("c")
    base_blk = wid * BLK_W
    pltpu.sync_copy(ids_hbm.at[pl.ds(base_blk, BLK_W)], idx_v)

    def step(g, carry):
        pltpu.async_copy(table_hbm.at[idx_v.at[g]], row_v, sem).wait()
        pltpu.sync_copy(row_v, out_hbm.at[pl.ds((base_blk + g) * CH, CH)])
        return carry

    lax.fori_loop(0, BLK_W, step, 0)


def kernel(input_ids, table):
    ids = input_ids.astype(jnp.int32).reshape(N_BLK, CH)
    out = _embed_lookup(ids, table)
    return out.reshape(B, L, HIDDEN)


# speedup vs baseline: 1.1130x; 1.1130x over previous
import functools

import jax
import jax.numpy as jnp
from jax import lax
from jax.experimental import pallas as pl
from jax.experimental.pallas import tpu as pltpu
from jax.experimental.pallas import tpu_sc as plsc

VOCAB = 1000000
HIDDEN = 64
B = 4096
L = 200

NC = 2
NS = 16
NW = NC * NS

N = B * L
CH = 128
N_BLK = N // CH
BLK_W = N_BLK // NW
G = 5
NGRP = BLK_W // G
GROWS = G * CH

_mesh = plsc.VectorSubcoreMesh(
    core_axis_name="c", subcore_axis_name="s", num_cores=NC, num_subcores=NS
)


@functools.partial(
    pl.kernel,
    out_type=jax.ShapeDtypeStruct((N, HIDDEN), jnp.float32),
    mesh=_mesh,
    scratch_types=[
        pltpu.VMEM((BLK_W, CH), jnp.int32),
        pltpu.VMEM((GROWS, HIDDEN), jnp.float32),
        pltpu.VMEM((GROWS, HIDDEN), jnp.float32),
        pltpu.SemaphoreType.DMA,
        pltpu.SemaphoreType.DMA,
        pltpu.SemaphoreType.DMA,
        pltpu.SemaphoreType.DMA,
    ],
    compiler_params=pltpu.CompilerParams(use_tc_tiling_on_sc=False),
)
def _embed_lookup(ids_hbm, table_hbm, out_hbm, idx_v, buf0, buf1,
                  sg0, sg1, so0, so1):
    wid = lax.axis_index("s") * NC + lax.axis_index("c")
    base_blk = wid * BLK_W
    bufs = (buf0, buf1)
    sgs = (sg0, sg1)
    sos = (so0, so1)

    pltpu.sync_copy(ids_hbm.at[pl.ds(base_blk, BLK_W)], idx_v)

    def fire(t, b):
        for j in range(G):
            pltpu.async_copy(
                table_hbm.at[idx_v.at[t * G + j]],
                bufs[b].at[pl.ds(j * CH, CH)],
                sgs[b],
            )

    def drain_gathers(b):
        pltpu.make_async_copy(
            out_hbm.at[pl.ds(0, GROWS)], bufs[b], sgs[b]
        ).wait()

    def start_out(t, b):
        pltpu.async_copy(
            bufs[b], out_hbm.at[pl.ds((base_blk + t * G) * CH, GROWS)], sos[b]
        )

    def wait_out(b):
        pltpu.make_async_copy(
            bufs[b], out_hbm.at[pl.ds(0, GROWS)], sos[b]
        ).wait()

    fire(0, 0)
    fire(1, 1)
    drain_gathers(0)
    start_out(0, 0)

    def step(k, carry):
        t = 1 + 2 * k
        wait_out(0)
        fire(t + 1, 0)
        drain_gathers(1)
        start_out(t, 1)
        wait_out(1)
        fire(t + 2, 1)
        drain_gathers(0)
        start_out(t + 1, 0)
        return carry

    lax.fori_loop(0, (NGRP - 2) // 2, step, 0)

    drain_gathers(1)
    start_out(NGRP - 1, 1)
    wait_out(0)
    wait_out(1)


def kernel(input_ids, table):
    ids = input_ids.astype(jnp.int32).reshape(N_BLK, CH)
    out = _embed_lookup(ids, table)
    return out.reshape(B, L, HIDDEN)
